# baseline (speedup 1.0000x reference)
import functools
import math

import jax
import jax.numpy as jnp
from jax.experimental import pallas as pl
from jax.experimental.pallas import tpu as pltpu

N = 10000
E = 320000
F = 128
H = 8
DH = F // H
NRBF = 16
ZMAX = 16
CUT_UPPER = 5.0

EB = 1280
NB = 40


def _h_body(x_ref, w_ref, o_ref):
    o_ref[...] = jnp.dot(x_ref[...], w_ref[...], preferred_element_type=jnp.float32)


def _wc_body(ea_ref, ew_ref, f1_ref, f1b_ref, f2_ref, f2b_ref, o_ref):
    t = jnp.dot(ea_ref[...], f1_ref[...], preferred_element_type=jnp.float32) + f1b_ref[...]
    t = t * jax.nn.sigmoid(t)
    t = jnp.dot(t, f2_ref[...], preferred_element_type=jnp.float32) + f2b_ref[...]
    w = ew_ref[...]
    c = 0.5 * (jnp.cos(w * (math.pi / CUT_UPPER)) + 1.0)
    c = c * (w < CUT_UPPER).astype(jnp.float32)
    o_ref[...] = t * c


KHALF = (N * ZMAX) // 2


def _scatter_body(base, src_ref, dst_ref, z_ref, h_ref, wc_ref, zc_ref,
                  y_ref, cnt_ref):
    @pl.when(pl.program_id(0) == 0)
    def _init():
        y_ref[...] = jnp.zeros_like(y_ref)

    def body(i, _):
        s = src_ref[0, i]
        d = dst_ref[0, i]
        zv = z_ref[0, s]
        k = d * ZMAX + zv - base

        @pl.when(jnp.logical_and(k >= 0, k < KHALF))
        def _apply():
            hrow = h_ref[pl.ds(s, 1), :]
            wrow = wc_ref[pl.ds(i, 1), :]
            yrow = y_ref[pl.ds(k, 1), :]
            y_ref[pl.ds(k, 1), :] = yrow + hrow * wrow
            cnt_ref[0, k] = 1.0

        return 0

    jax.lax.fori_loop(0, EB, body, 0)


def _attn_body(y_ref, cnt_ref, cnt2_ref, qw_ref, qb_ref, kw_ref, kb_ref, vw_ref, vb_ref,
               ow_ref, ob_ref, lw_ref, lb_ref, o_ref):
    yb = y_ref[...]
    q = jnp.dot(yb, qw_ref[...], preferred_element_type=jnp.float32) + qb_ref[...]
    k = jnp.dot(yb, kw_ref[...], preferred_element_type=jnp.float32) + kb_ref[...]
    v = jnp.dot(yb, vw_ref[...], preferred_element_type=jnp.float32) + vb_ref[...]
    mask = cnt_ref[...] > 0.0
    outs = []
    for hh in range(H):
        sl = slice(hh * DH, (hh + 1) * DH)
        qh = q[:, sl].reshape(NB, ZMAX, DH)
        kh = k[:, sl].reshape(NB, ZMAX, DH)
        vh = v[:, sl].reshape(NB, ZMAX, DH)
        lg = (qh[:, :, None, :] * kh[:, None, :, :]).sum(-1)
        lg = jnp.where(mask[:, None, :], lg, -1e9)
        m = lg.max(axis=-1, keepdims=True)
        e = jnp.exp(lg - m)
        a = e / e.sum(axis=-1, keepdims=True)
        oh = (a[:, :, :, None] * vh[:, None, :, :]).sum(axis=2)
        outs.append(oh)
    out = jnp.concatenate(outs, axis=-1)
    out = jnp.dot(out.reshape(NB * ZMAX, F), ow_ref[...],
                  preferred_element_type=jnp.float32) + ob_ref[...]
    out = jnp.where(cnt2_ref[...] > 0.0, out, 0.0)
    node = out.reshape(NB, ZMAX, F).sum(axis=1)
    node = node * jax.nn.sigmoid(node)
    o_ref[...] = jnp.dot(node, lw_ref[...], preferred_element_type=jnp.float32) + lb_ref[...]


@jax.jit
def kernel(x, z, edge_index, edge_weight, edge_attr,
           lin1_W, fn1_W, fn1_b, fn2_W, fn2_b,
           q_W, q_b, k_W, k_b, v_W, v_b, o_W, o_b,
           lin_W, lin_b):
    f32 = jnp.float32
    src = edge_index[0].reshape(1, E).astype(jnp.int32)
    dst = edge_index[1].reshape(1, E).astype(jnp.int32)
    z2 = z.reshape(1, N).astype(jnp.int32)
    ew = edge_weight.reshape(E, 1)
    row = lambda b: b.reshape(1, F)

    h = pl.pallas_call(
        _h_body,
        out_shape=jax.ShapeDtypeStruct((N, F), f32),
    )(x, lin1_W.T)

    full = lambda shape: pl.BlockSpec(shape, lambda i: (0, 0))
    wc = pl.pallas_call(
        _wc_body,
        grid=(E // EB,),
        in_specs=[
            pl.BlockSpec((EB, NRBF), lambda i: (i, 0)),
            pl.BlockSpec((EB, 1), lambda i: (i, 0)),
            full((NRBF, F)), full((1, F)), full((F, F)), full((1, F)),
        ],
        out_specs=pl.BlockSpec((EB, F), lambda i: (i, 0)),
        out_shape=jax.ShapeDtypeStruct((E, F), f32),
    )(edge_attr, ew, fn1_W.T, row(fn1_b), fn2_W.T, row(fn2_b))

    halves = []
    for p in range(2):
        yh, ch = pl.pallas_call(
            functools.partial(_scatter_body, p * KHALF),
            grid=(E // EB,),
            in_specs=[
                pl.BlockSpec((1, EB), lambda i: (0, i),
                             memory_space=pltpu.SMEM),
                pl.BlockSpec((1, EB), lambda i: (0, i),
                             memory_space=pltpu.SMEM),
                pl.BlockSpec((1, N), lambda i: (0, 0),
                             memory_space=pltpu.SMEM),
                full((N, F)),
                pl.BlockSpec((EB, F), lambda i: (i, 0)),
                pl.BlockSpec((1, KHALF), lambda i: (0, 0),
                             memory_space=pltpu.SMEM),
            ],
            out_specs=[full((KHALF, F)),
                       pl.BlockSpec((1, KHALF), lambda i: (0, 0),
                                    memory_space=pltpu.SMEM)],
            out_shape=[jax.ShapeDtypeStruct((KHALF, F), f32),
                       jax.ShapeDtypeStruct((1, KHALF), f32)],
            input_output_aliases={5: 1},
        )(src, dst, z2, h, wc, jnp.zeros((1, KHALF), f32))
        halves.append((yh, ch))
    y_acc = jnp.concatenate([halves[0][0], halves[1][0]], axis=0)
    cnt = jnp.concatenate([halves[0][1], halves[1][1]], axis=1)
    cnt = cnt.reshape(N, ZMAX)

    out = pl.pallas_call(
        _attn_body,
        grid=(N // NB,),
        in_specs=[
            pl.BlockSpec((NB * ZMAX, F), lambda i: (i, 0)),
            pl.BlockSpec((NB, ZMAX), lambda i: (i, 0)),
            pl.BlockSpec((NB * ZMAX, 1), lambda i: (i, 0)),
            full((F, F)), full((1, F)), full((F, F)), full((1, F)),
            full((F, F)), full((1, F)), full((F, F)), full((1, F)),
            full((F, F)), full((1, F)),
        ],
        out_specs=pl.BlockSpec((NB, F), lambda i: (i, 0)),
        out_shape=jax.ShapeDtypeStruct((N, F), f32),
    )(y_acc, cnt, cnt.reshape(N * ZMAX, 1), q_W.T, row(q_b), k_W.T, row(k_b), v_W.T, row(v_b),
      o_W.T, row(o_b), lin_W.T, row(lin_b))
    return out


# speedup vs baseline: 1.2153x; 1.2153x over previous
import functools
import math

import jax
import jax.numpy as jnp
from jax.experimental import pallas as pl
from jax.experimental.pallas import tpu as pltpu

N = 10000
E = 320000
F = 128
H = 8
DH = F // H
NRBF = 16
ZMAX = 16
CUT_UPPER = 5.0

EB = 2560
NB = 40


def _h_body(x_ref, w_ref, o_ref):
    o_ref[...] = jnp.dot(x_ref[...], w_ref[...], preferred_element_type=jnp.float32)


def _wc_body(ea_ref, ew_ref, f1_ref, f1b_ref, f2_ref, f2b_ref, o_ref):
    t = jnp.dot(ea_ref[...], f1_ref[...], preferred_element_type=jnp.float32) + f1b_ref[...]
    t = t * jax.nn.sigmoid(t)
    t = jnp.dot(t, f2_ref[...], preferred_element_type=jnp.float32) + f2b_ref[...]
    w = ew_ref[...]
    c = 0.5 * (jnp.cos(w * (math.pi / CUT_UPPER)) + 1.0)
    c = c * (w < CUT_UPPER).astype(jnp.float32)
    o_ref[...] = t * c


KHALF = (N * ZMAX) // 2


def _scatter_body(base, src_ref, dst_ref, z_ref, h_ref, wc_ref, zc_ref,
                  y_ref, cnt_ref):
    @pl.when(pl.program_id(0) == 0)
    def _init():
        y_ref[...] = jnp.zeros_like(y_ref)

    def body(i, _):
        s = src_ref[0, i]
        d = dst_ref[0, i]
        zv = z_ref[0, s]
        k = d * ZMAX + zv - base

        @pl.when(jnp.logical_and(k >= 0, k < KHALF))
        def _apply():
            hrow = h_ref[pl.ds(s, 1), :]
            wrow = wc_ref[pl.ds(i, 1), :]
            yrow = y_ref[pl.ds(k, 1), :]
            y_ref[pl.ds(k, 1), :] = yrow + hrow * wrow
            cnt_ref[0, k] = 1.0

        return 0

    jax.lax.fori_loop(0, EB, body, 0, unroll=8)


def _attn_body(y_ref, cnt_ref, cnt2_ref, qw_ref, qb_ref, kw_ref, kb_ref, vw_ref, vb_ref,
               ow_ref, ob_ref, lw_ref, lb_ref, o_ref):
    yb = y_ref[...]
    q = jnp.dot(yb, qw_ref[...], preferred_element_type=jnp.float32) + qb_ref[...]
    k = jnp.dot(yb, kw_ref[...], preferred_element_type=jnp.float32) + kb_ref[...]
    v = jnp.dot(yb, vw_ref[...], preferred_element_type=jnp.float32) + vb_ref[...]
    mask = cnt_ref[...] > 0.0
    outs = []
    for hh in range(H):
        sl = slice(hh * DH, (hh + 1) * DH)
        qh = q[:, sl].reshape(NB, ZMAX, DH)
        kh = k[:, sl].reshape(NB, ZMAX, DH)
        vh = v[:, sl].reshape(NB, ZMAX, DH)
        lg = (qh[:, :, None, :] * kh[:, None, :, :]).sum(-1)
        lg = jnp.where(mask[:, None, :], lg, -1e9)
        m = lg.max(axis=-1, keepdims=True)
        e = jnp.exp(lg - m)
        a = e / e.sum(axis=-1, keepdims=True)
        oh = (a[:, :, :, None] * vh[:, None, :, :]).sum(axis=2)
        outs.append(oh)
    out = jnp.concatenate(outs, axis=-1)
    out = jnp.dot(out.reshape(NB * ZMAX, F), ow_ref[...],
                  preferred_element_type=jnp.float32) + ob_ref[...]
    out = jnp.where(cnt2_ref[...] > 0.0, out, 0.0)
    node = out.reshape(NB, ZMAX, F).sum(axis=1)
    node = node * jax.nn.sigmoid(node)
    o_ref[...] = jnp.dot(node, lw_ref[...], preferred_element_type=jnp.float32) + lb_ref[...]


@jax.jit
def kernel(x, z, edge_index, edge_weight, edge_attr,
           lin1_W, fn1_W, fn1_b, fn2_W, fn2_b,
           q_W, q_b, k_W, k_b, v_W, v_b, o_W, o_b,
           lin_W, lin_b):
    f32 = jnp.float32
    src = edge_index[0].reshape(1, E).astype(jnp.int32)
    dst = edge_index[1].reshape(1, E).astype(jnp.int32)
    z2 = z.reshape(1, N).astype(jnp.int32)
    ew = edge_weight.reshape(E, 1)
    row = lambda b: b.reshape(1, F)

    h = pl.pallas_call(
        _h_body,
        out_shape=jax.ShapeDtypeStruct((N, F), f32),
    )(x, lin1_W.T)

    full = lambda shape: pl.BlockSpec(shape, lambda i: (0, 0))
    wc = pl.pallas_call(
        _wc_body,
        grid=(E // EB,),
        in_specs=[
            pl.BlockSpec((EB, NRBF), lambda i: (i, 0)),
            pl.BlockSpec((EB, 1), lambda i: (i, 0)),
            full((NRBF, F)), full((1, F)), full((F, F)), full((1, F)),
        ],
        out_specs=pl.BlockSpec((EB, F), lambda i: (i, 0)),
        out_shape=jax.ShapeDtypeStruct((E, F), f32),
    )(edge_attr, ew, fn1_W.T, row(fn1_b), fn2_W.T, row(fn2_b))

    halves = []
    for p in range(2):
        yh, ch = pl.pallas_call(
            functools.partial(_scatter_body, p * KHALF),
            grid=(E // EB,),
            in_specs=[
                pl.BlockSpec((1, EB), lambda i: (0, i),
                             memory_space=pltpu.SMEM),
                pl.BlockSpec((1, EB), lambda i: (0, i),
                             memory_space=pltpu.SMEM),
                pl.BlockSpec((1, N), lambda i: (0, 0),
                             memory_space=pltpu.SMEM),
                full((N, F)),
                pl.BlockSpec((EB, F), lambda i: (i, 0)),
                pl.BlockSpec((1, KHALF), lambda i: (0, 0),
                             memory_space=pltpu.SMEM),
            ],
            out_specs=[full((KHALF, F)),
                       pl.BlockSpec((1, KHALF), lambda i: (0, 0),
                                    memory_space=pltpu.SMEM)],
            out_shape=[jax.ShapeDtypeStruct((KHALF, F), f32),
                       jax.ShapeDtypeStruct((1, KHALF), f32)],
            input_output_aliases={5: 1},
        )(src, dst, z2, h, wc, jnp.zeros((1, KHALF), f32))
        halves.append((yh, ch))
    y_acc = jnp.concatenate([halves[0][0], halves[1][0]], axis=0)
    cnt = jnp.concatenate([halves[0][1], halves[1][1]], axis=1)
    cnt = cnt.reshape(N, ZMAX)

    out = pl.pallas_call(
        _attn_body,
        grid=(N // NB,),
        in_specs=[
            pl.BlockSpec((NB * ZMAX, F), lambda i: (i, 0)),
            pl.BlockSpec((NB, ZMAX), lambda i: (i, 0)),
            pl.BlockSpec((NB * ZMAX, 1), lambda i: (i, 0)),
            full((F, F)), full((1, F)), full((F, F)), full((1, F)),
            full((F, F)), full((1, F)), full((F, F)), full((1, F)),
            full((F, F)), full((1, F)),
        ],
        out_specs=pl.BlockSpec((NB, F), lambda i: (i, 0)),
        out_shape=jax.ShapeDtypeStruct((N, F), f32),
    )(y_acc, cnt, cnt.reshape(N * ZMAX, 1), q_W.T, row(q_b), k_W.T, row(k_b), v_W.T, row(v_b),
      o_W.T, row(o_b), lin_W.T, row(lin_b))
    return out
